# baseline (speedup 1.0000x reference)
import functools

import jax
import jax.numpy as jnp
from jax import lax
from jax.experimental import pallas as pl
from jax.experimental.pallas import tpu as pltpu
from jax.experimental.pallas import tpu_sc as plsc

E = 800000
N = 50000
BE = 4000
BN = 2000

_PREC = lax.Precision.DEFAULT

_NTILES = 16
_EPT = E // _NTILES
_G = 400
_GSUB = 100
_NCH = 500
_NCHUNKS = N // _NCH


def _dot(a, b):
    return lax.dot_general(a, b, (((1,), (0,)), ((), ())),
                           precision=_PREC, preferred_element_type=jnp.float32)


def _edge_block_kernel(xi_ref, xj_ref, ii_ref, ij_ref, ea_ref, aea_ref,
                       w_refs, ue_ref, msg_ref):
    (eu_w1, eu_b1, eu_w2, eu_b2, eu_w3, eu_b3,
     f_w1, f_b1, f_w2, f_b2,
     p_w1, p_b1, p_w2, p_b2) = w_refs
    x_i = xi_ref[...]
    x_j = xj_ref[...]
    init_i = ii_ref[...]
    init_j = ij_ref[...]
    ea = ea_ref[...]
    aea = aea_ref[...]

    feats = jnp.concatenate([x_i, x_j, ea, aea], axis=1)
    h = jax.nn.relu(_dot(feats, eu_w1[...]) + eu_b1[...])
    h = jax.nn.relu(_dot(h, eu_w2[...]) + eu_b2[...])
    ue = _dot(h, eu_w3[...]) + eu_b3[...]
    ue_ref[...] = ue

    pfeat = jnp.concatenate([x_j, ue, init_j], axis=1)
    hp = jax.nn.relu(_dot(pfeat, p_w1[...]) + p_b1[...])
    msg_ref[:, :128] = _dot(hp, p_w2[...]) + p_b2[...]

    ffeat = jnp.concatenate([x_i, ue, init_i], axis=1)
    hf = jax.nn.relu(_dot(ffeat, f_w1[...]) + f_b1[...])
    msg_ref[:, 128:] = _dot(hf, f_w2[...]) + f_b2[...]


def _node_block_kernel(msg_ref, w_refs, out_ref):
    (w1, b1, w2, b2, w3, b3) = w_refs
    m = msg_ref[...]
    h = jax.nn.relu(_dot(m, w1[...]) + b1[...])
    h = jax.nn.relu(_dot(h, w2[...]) + b2[...])
    out_ref[...] = _dot(h, w3[...]) + b3[...]


def _edge_stage(x_i, x_j, init_i, init_j, edge_attr, att_edge_attr, wflat):
    nblk = E // BE
    eb = lambda i: (i, 0)
    wspec = [pl.BlockSpec(w.shape, lambda i, nd=w.ndim: (0,) * nd) for w in wflat]
    grid_spec = pltpu.PrefetchScalarGridSpec(
        num_scalar_prefetch=0,
        grid=(nblk,),
        in_specs=[
            pl.BlockSpec((BE, 96), eb),
            pl.BlockSpec((BE, 96), eb),
            pl.BlockSpec((BE, 96), eb),
            pl.BlockSpec((BE, 96), eb),
            pl.BlockSpec((BE, 64), eb),
            pl.BlockSpec((BE, 64), eb),
            wspec,
        ],
        out_specs=[
            pl.BlockSpec((BE, 64), eb),
            pl.BlockSpec((BE, 256), eb),
        ],
    )
    return pl.pallas_call(
        _edge_block_kernel,
        grid_spec=grid_spec,
        out_shape=[
            jax.ShapeDtypeStruct((E, 64), jnp.float32),
            jax.ShapeDtypeStruct((E, 256), jnp.float32),
        ],
    )(x_i, x_j, init_i, init_j, edge_attr, att_edge_attr, wflat)


def _node_stage(messages, wflat):
    nblk = N // BN
    wspec = [pl.BlockSpec(w.shape, lambda i, nd=w.ndim: (0,) * nd) for w in wflat]
    grid_spec = pltpu.PrefetchScalarGridSpec(
        num_scalar_prefetch=0,
        grid=(nblk,),
        in_specs=[pl.BlockSpec((BN, 256), lambda i: (i, 0)), wspec],
        out_specs=pl.BlockSpec((BN, 96), lambda i: (i, 0)),
    )
    return pl.pallas_call(
        _node_block_kernel,
        grid_spec=grid_spec,
        out_shape=jax.ShapeDtypeStruct((N, 96), jnp.float32),
    )(messages, wflat)


def _sc_scatter_kernel(eidx3, msgs8, out8, idxb, msgb, stage, table):
    c = lax.axis_index("c")
    s = lax.axis_index("s")
    zero16 = jnp.zeros((16,), jnp.float32)

    for q in range(4):
        c4 = c * 4 + q

        def _zero_stage(i, _):
            stage[i, pl.ds(0, 16)] = zero16
            stage[i, pl.ds(16, 16)] = zero16
            return _
        lax.fori_loop(0, _NCH, _zero_stage, None)

        def _zero_table(m, _):
            k = s + m * _NTILES

            @pl.when(k < _NCHUNKS)
            def _():
                pltpu.sync_copy(stage, table.at[pl.ds(k * _NCH, _NCH)])
            return _
        lax.fori_loop(0, (_NCHUNKS + _NTILES - 1) // _NTILES, _zero_table, None)
        plsc.subcore_barrier()

        def _chunk(g, _):
            rowbase = s * (_EPT // _GSUB) + g * (_G // _GSUB)
            pltpu.sync_copy(eidx3.at[c, pl.ds(rowbase, _G // _GSUB)], idxb)
            ebase = s * _EPT + g * _G
            pltpu.sync_copy(msgs8.at[pl.ds(ebase, _G), c4], msgb)
            for j in range(_G // _GSUB):
                pltpu.sync_copy(msgb.at[pl.ds(j * _GSUB, _GSUB)],
                                table.at[idxb.at[j]], add=True)
            return _
        lax.fori_loop(0, _EPT // _G, _chunk, None)
        plsc.subcore_barrier()

        def _writeout(m, _):
            k = s + m * _NTILES

            @pl.when(k < _NCHUNKS)
            def _():
                pltpu.sync_copy(table.at[pl.ds(k * _NCH, _NCH)], stage)
                pltpu.sync_copy(stage, out8.at[pl.ds(k * _NCH, _NCH), c4])
            return _
        lax.fori_loop(0, (_NCHUNKS + _NTILES - 1) // _NTILES, _writeout, None)
        plsc.subcore_barrier()


def _scatter_stage(msgs, edge_index):
    eidx3 = jnp.stack([edge_index[1], edge_index[0]]).reshape(2, E // _GSUB, _GSUB)
    msgs8 = msgs.reshape(E, 8, 32)
    mesh = plsc.VectorSubcoreMesh(core_axis_name="c", subcore_axis_name="s")
    scatter = pl.kernel(
        _sc_scatter_kernel,
        mesh=mesh,
        compiler_params=pltpu.CompilerParams(use_tc_tiling_on_sc=False),
        out_type=jax.ShapeDtypeStruct((N, 8, 32), jnp.float32),
        scratch_types=[
            pltpu.VMEM((_G // _GSUB, _GSUB), jnp.int32),
            pltpu.VMEM((_G, 32), jnp.float32),
            pltpu.VMEM((_NCH, 32), jnp.float32),
            pltpu.VMEM_SHARED((N, 32), jnp.float32),
        ],
    )
    return scatter(eidx3, msgs8).reshape(N, 256)


def kernel(x, edge_attr, initial_x, att_edge_attr, params, edge_index):
    rows = edge_index[0]
    cols = edge_index[1]
    x_j = jnp.take(x, rows, axis=0)
    x_i = jnp.take(x, cols, axis=0)
    init_j = jnp.take(initial_x, rows, axis=0)
    init_i = jnp.take(initial_x, cols, axis=0)

    eu = params["edge_update"]
    fm = params["create_future_msgs"]
    pm = params["create_past_msgs"]
    cb = params["combine_future_past"]

    edge_w = (eu[0][0], eu[0][1], eu[1][0], eu[1][1], eu[2][0], eu[2][1],
              fm[0][0], fm[0][1], fm[1][0], fm[1][1],
              pm[0][0], pm[0][1], pm[1][0], pm[1][1])
    ue, msgs = _edge_stage(x_i, x_j, init_i, init_j,
                           edge_attr, att_edge_attr, list(edge_w))

    messages = _scatter_stage(msgs, edge_index)

    node_w = [cb[0][0], cb[0][1], cb[1][0], cb[1][1], cb[2][0], cb[2][1]]
    updated_nodes = _node_stage(messages, node_w)
    return (updated_nodes, ue)


# speedup vs baseline: 1.4830x; 1.4830x over previous
import functools

import jax
import jax.numpy as jnp
from jax import lax
from jax.experimental import pallas as pl
from jax.experimental.pallas import tpu as pltpu
from jax.experimental.pallas import tpu_sc as plsc

E = 800000
N = 50000
BE = 4000
BN = 2000

_PREC = lax.Precision.DEFAULT

_NTILES = 16
_EPT = E // _NTILES
_G = 200
_GSUB = 40
_NSUB = _G // _GSUB
_NITER = _EPT // _G
_NCH = 250
_NCHUNKS = N // _NCH


def _dot(a, b):
    return lax.dot_general(a, b, (((1,), (0,)), ((), ())),
                           precision=_PREC, preferred_element_type=jnp.float32)


def _edge_block_kernel(xi_ref, xj_ref, ii_ref, ij_ref, ea_ref, aea_ref,
                       w_refs, ue_ref, past_ref, fut_ref):
    (eu_w1, eu_b1, eu_w2, eu_b2, eu_w3, eu_b3,
     f_w1, f_b1, f_w2, f_b2,
     p_w1, p_b1, p_w2, p_b2) = w_refs
    x_i = xi_ref[...]
    x_j = xj_ref[...]
    init_i = ii_ref[...]
    init_j = ij_ref[...]
    ea = ea_ref[...]
    aea = aea_ref[...]

    feats = jnp.concatenate([x_i, x_j, ea, aea], axis=1)
    h = jax.nn.relu(_dot(feats, eu_w1[...]) + eu_b1[...])
    h = jax.nn.relu(_dot(h, eu_w2[...]) + eu_b2[...])
    ue = _dot(h, eu_w3[...]) + eu_b3[...]
    ue_ref[...] = ue

    pfeat = jnp.concatenate([x_j, ue, init_j], axis=1)
    hp = jax.nn.relu(_dot(pfeat, p_w1[...]) + p_b1[...])
    past_ref[...] = _dot(hp, p_w2[...]) + p_b2[...]

    ffeat = jnp.concatenate([x_i, ue, init_i], axis=1)
    hf = jax.nn.relu(_dot(ffeat, f_w1[...]) + f_b1[...])
    fut_ref[...] = _dot(hf, f_w2[...]) + f_b2[...]


def _node_block_kernel(mp_ref, mf_ref, w_refs, out_ref):
    (w1p, w1f, b1, w2, b2, w3, b3) = w_refs
    h = jax.nn.relu(_dot(mp_ref[...], w1p[...]) + _dot(mf_ref[...], w1f[...])
                    + b1[...])
    h = jax.nn.relu(_dot(h, w2[...]) + b2[...])
    out_ref[...] = _dot(h, w3[...]) + b3[...]


def _edge_stage(x_i, x_j, init_i, init_j, edge_attr, att_edge_attr, wflat):
    nblk = E // BE
    eb = lambda i: (i, 0)
    wspec = [pl.BlockSpec(w.shape, lambda i, nd=w.ndim: (0,) * nd) for w in wflat]
    grid_spec = pltpu.PrefetchScalarGridSpec(
        num_scalar_prefetch=0,
        grid=(nblk,),
        in_specs=[
            pl.BlockSpec((BE, 96), eb),
            pl.BlockSpec((BE, 96), eb),
            pl.BlockSpec((BE, 96), eb),
            pl.BlockSpec((BE, 96), eb),
            pl.BlockSpec((BE, 64), eb),
            pl.BlockSpec((BE, 64), eb),
            wspec,
        ],
        out_specs=[
            pl.BlockSpec((BE, 64), eb),
            pl.BlockSpec((BE, 128), eb),
            pl.BlockSpec((BE, 128), eb),
        ],
    )
    return pl.pallas_call(
        _edge_block_kernel,
        grid_spec=grid_spec,
        out_shape=[
            jax.ShapeDtypeStruct((E, 64), jnp.float32),
            jax.ShapeDtypeStruct((E, 128), jnp.float32),
            jax.ShapeDtypeStruct((E, 128), jnp.float32),
        ],
    )(x_i, x_j, init_i, init_j, edge_attr, att_edge_attr, wflat)


def _node_stage(mp, mf, wflat):
    nblk = N // BN
    wspec = [pl.BlockSpec(w.shape, lambda i, nd=w.ndim: (0,) * nd) for w in wflat]
    grid_spec = pltpu.PrefetchScalarGridSpec(
        num_scalar_prefetch=0,
        grid=(nblk,),
        in_specs=[pl.BlockSpec((BN, 128), lambda i: (i, 0)),
                  pl.BlockSpec((BN, 128), lambda i: (i, 0)),
                  wspec],
        out_specs=pl.BlockSpec((BN, 96), lambda i: (i, 0)),
    )
    return pl.pallas_call(
        _node_block_kernel,
        grid_spec=grid_spec,
        out_shape=jax.ShapeDtypeStruct((N, 96), jnp.float32),
    )(mp, mf, wflat)


def _sc_scatter_one_core(msg_hbm, idx_hbm, out_hbm, idxb, msgb, stage, table,
                         sem, s):
    zero16 = jnp.zeros((16,), jnp.float32)

    def _issue(g, slot, q):
        ebase = s * _EPT + g * _G
        for j in range(_NSUB):
            pltpu.async_copy(idx_hbm.at[pl.ds(ebase + j * _GSUB, _GSUB)],
                             idxb.at[slot, j], sem.at[slot])
        pltpu.async_copy(msg_hbm.at[pl.ds(ebase, _G), pl.ds(q * 32, 32)],
                         msgb.at[slot], sem.at[slot])

    def _drain(g, slot, q):
        ebase = s * _EPT + g * _G
        for j in range(_NSUB):
            pltpu.make_async_copy(idx_hbm.at[pl.ds(ebase + j * _GSUB, _GSUB)],
                                  idxb.at[slot, j], sem.at[slot]).wait()
        pltpu.make_async_copy(msg_hbm.at[pl.ds(ebase, _G), pl.ds(q * 32, 32)],
                              msgb.at[slot], sem.at[slot]).wait()

    for q in range(4):
        def _zero_stage(i, carry):
            stage[i, pl.ds(0, 16)] = zero16
            stage[i, pl.ds(16, 16)] = zero16
            return carry
        lax.fori_loop(0, _NCH, _zero_stage, 0)

        def _zero_table(m, carry):
            k = s + m * _NTILES

            @pl.when(k < _NCHUNKS)
            def _():
                pltpu.sync_copy(stage, table.at[pl.ds(k * _NCH, _NCH)])
            return carry
        lax.fori_loop(0, (_NCHUNKS + _NTILES - 1) // _NTILES, _zero_table, 0)
        plsc.subcore_barrier()

        _issue(0, 0, q)

        def _chunk(g, carry):
            slot = lax.rem(g, 2)

            @pl.when(g + 1 < _NITER)
            def _():
                _issue(g + 1, 1 - slot, q)
            _drain(g, slot, q)
            for j in range(_NSUB):
                pltpu.sync_copy(msgb.at[slot, pl.ds(j * _GSUB, _GSUB)],
                                table.at[idxb.at[slot, j]], add=True)
            return carry
        lax.fori_loop(0, _NITER, _chunk, 0)
        plsc.subcore_barrier()

        def _writeout(m, carry):
            k = s + m * _NTILES

            @pl.when(k < _NCHUNKS)
            def _():
                pltpu.sync_copy(table.at[pl.ds(k * _NCH, _NCH)], stage)
                pltpu.sync_copy(stage, out_hbm.at[pl.ds(k * _NCH, _NCH),
                                                  pl.ds(q * 32, 32)])
            return carry
        lax.fori_loop(0, (_NCHUNKS + _NTILES - 1) // _NTILES, _writeout, 0)
        plsc.subcore_barrier()


def _sc_scatter_kernel(past_hbm, fut_hbm, cols_hbm, rows_hbm,
                       outp_hbm, outf_hbm, idxb, msgb, stage, table, sem):
    c = lax.axis_index("c")
    s = lax.axis_index("s")

    @pl.when(c == 0)
    def _():
        _sc_scatter_one_core(past_hbm, cols_hbm, outp_hbm,
                             idxb, msgb, stage, table, sem, s)

    @pl.when(c == 1)
    def _():
        _sc_scatter_one_core(fut_hbm, rows_hbm, outf_hbm,
                             idxb, msgb, stage, table, sem, s)


def _scatter_stage(past, fut, cols, rows):
    mesh = plsc.VectorSubcoreMesh(core_axis_name="c", subcore_axis_name="s")
    scatter = pl.kernel(
        _sc_scatter_kernel,
        mesh=mesh,
        compiler_params=pltpu.CompilerParams(use_tc_tiling_on_sc=False),
        out_type=[jax.ShapeDtypeStruct((N, 128), jnp.float32),
                  jax.ShapeDtypeStruct((N, 128), jnp.float32)],
        scratch_types=[
            pltpu.VMEM((2, _NSUB, _GSUB), jnp.int32),
            pltpu.VMEM((2, _G, 32), jnp.float32),
            pltpu.VMEM((_NCH, 32), jnp.float32),
            pltpu.VMEM_SHARED((N, 32), jnp.float32),
            pltpu.SemaphoreType.DMA((2,)),
        ],
    )
    return scatter(past, fut, cols, rows)


def kernel(x, edge_attr, initial_x, att_edge_attr, params, edge_index):
    rows = edge_index[0]
    cols = edge_index[1]
    x_j = jnp.take(x, rows, axis=0)
    x_i = jnp.take(x, cols, axis=0)
    init_j = jnp.take(initial_x, rows, axis=0)
    init_i = jnp.take(initial_x, cols, axis=0)

    eu = params["edge_update"]
    fm = params["create_future_msgs"]
    pm = params["create_past_msgs"]
    cb = params["combine_future_past"]

    edge_w = (eu[0][0], eu[0][1], eu[1][0], eu[1][1], eu[2][0], eu[2][1],
              fm[0][0], fm[0][1], fm[1][0], fm[1][1],
              pm[0][0], pm[0][1], pm[1][0], pm[1][1])
    ue, past, fut = _edge_stage(x_i, x_j, init_i, init_j,
                                edge_attr, att_edge_attr, list(edge_w))

    mp, mf = _scatter_stage(past, fut, cols, rows)

    node_w = [cb[0][0][:128], cb[0][0][128:], cb[0][1],
              cb[1][0], cb[1][1], cb[2][0], cb[2][1]]
    updated_nodes = _node_stage(mp, mf, node_w)
    return (updated_nodes, ue)


# speedup vs baseline: 3.3719x; 2.2738x over previous
import functools

import jax
import jax.numpy as jnp
from jax import lax
from jax.experimental import pallas as pl
from jax.experimental.pallas import tpu as pltpu
from jax.experimental.pallas import tpu_sc as plsc

E = 800000
N = 50000
BE = 4000
BN = 2000

_PREC = lax.Precision.DEFAULT

_NTILES = 16
_EPT = E // _NTILES
_G = 200
_GSUB = 40
_NSUB = _G // _GSUB
_NITER = _EPT // _G
_NCH = 250
_NCHUNKS = N // _NCH


def _dot(a, b):
    return lax.dot_general(a, b, (((1,), (0,)), ((), ())),
                           precision=_PREC, preferred_element_type=jnp.float32)


def _edge_block_kernel(xi_ref, xj_ref, ii_ref, ij_ref, ea_ref, aea_ref,
                       w_refs, ue_ref, past_ref, fut_ref):
    (eu_w1, eu_b1, eu_w2, eu_b2, eu_w3, eu_b3,
     f_w1, f_b1, f_w2, f_b2,
     p_w1, p_b1, p_w2, p_b2) = w_refs
    x_i = xi_ref[:, :96]
    x_j = xj_ref[:, :96]
    init_i = ii_ref[:, :96]
    init_j = ij_ref[:, :96]
    ea = ea_ref[...]
    aea = aea_ref[...]

    feats = jnp.concatenate([x_i, x_j, ea, aea], axis=1)
    h = jax.nn.relu(_dot(feats, eu_w1[...]) + eu_b1[...])
    h = jax.nn.relu(_dot(h, eu_w2[...]) + eu_b2[...])
    ue = _dot(h, eu_w3[...]) + eu_b3[...]
    ue_ref[...] = ue

    pfeat = jnp.concatenate([x_j, ue, init_j], axis=1)
    hp = jax.nn.relu(_dot(pfeat, p_w1[...]) + p_b1[...])
    past_ref[...] = _dot(hp, p_w2[...]) + p_b2[...]

    ffeat = jnp.concatenate([x_i, ue, init_i], axis=1)
    hf = jax.nn.relu(_dot(ffeat, f_w1[...]) + f_b1[...])
    fut_ref[...] = _dot(hf, f_w2[...]) + f_b2[...]


def _node_block_kernel(mp_ref, mf_ref, w_refs, out_ref):
    (w1p, w1f, b1, w2, b2, w3, b3) = w_refs
    h = jax.nn.relu(_dot(mp_ref[...], w1p[...]) + _dot(mf_ref[...], w1f[...])
                    + b1[...])
    h = jax.nn.relu(_dot(h, w2[...]) + b2[...])
    out_ref[...] = _dot(h, w3[...]) + b3[...]


def _edge_stage(x_i, x_j, init_i, init_j, edge_attr, att_edge_attr, wflat):
    nblk = E // BE
    eb = lambda i: (i, 0)
    wspec = [pl.BlockSpec(w.shape, lambda i, nd=w.ndim: (0,) * nd) for w in wflat]
    grid_spec = pltpu.PrefetchScalarGridSpec(
        num_scalar_prefetch=0,
        grid=(nblk,),
        in_specs=[
            pl.BlockSpec((BE, 128), eb),
            pl.BlockSpec((BE, 128), eb),
            pl.BlockSpec((BE, 128), eb),
            pl.BlockSpec((BE, 128), eb),
            pl.BlockSpec((BE, 64), eb),
            pl.BlockSpec((BE, 64), eb),
            wspec,
        ],
        out_specs=[
            pl.BlockSpec((BE, 64), eb),
            pl.BlockSpec((BE, 128), eb),
            pl.BlockSpec((BE, 128), eb),
        ],
    )
    return pl.pallas_call(
        _edge_block_kernel,
        grid_spec=grid_spec,
        out_shape=[
            jax.ShapeDtypeStruct((E, 64), jnp.float32),
            jax.ShapeDtypeStruct((E, 128), jnp.float32),
            jax.ShapeDtypeStruct((E, 128), jnp.float32),
        ],
    )(x_i, x_j, init_i, init_j, edge_attr, att_edge_attr, wflat)


def _node_stage(mp, mf, wflat):
    nblk = N // BN
    wspec = [pl.BlockSpec(w.shape, lambda i, nd=w.ndim: (0,) * nd) for w in wflat]
    grid_spec = pltpu.PrefetchScalarGridSpec(
        num_scalar_prefetch=0,
        grid=(nblk,),
        in_specs=[pl.BlockSpec((BN, 128), lambda i: (i, 0)),
                  pl.BlockSpec((BN, 128), lambda i: (i, 0)),
                  wspec],
        out_specs=pl.BlockSpec((BN, 96), lambda i: (i, 0)),
    )
    return pl.pallas_call(
        _node_block_kernel,
        grid_spec=grid_spec,
        out_shape=jax.ShapeDtypeStruct((N, 96), jnp.float32),
    )(mp, mf, wflat)


def _sc_scatter_one_core(msg_hbm, idx_hbm, out_hbm, idxb, msgb, stage, table,
                         sem, s):
    zero16 = jnp.zeros((16,), jnp.float32)

    def _issue(g, slot, q):
        ebase = s * _EPT + g * _G
        for j in range(_NSUB):
            pltpu.async_copy(idx_hbm.at[pl.ds(ebase + j * _GSUB, _GSUB)],
                             idxb.at[slot, j], sem.at[slot])
        pltpu.async_copy(msg_hbm.at[pl.ds(ebase, _G), pl.ds(q * 32, 32)],
                         msgb.at[slot], sem.at[slot])

    def _drain(g, slot, q):
        ebase = s * _EPT + g * _G
        for j in range(_NSUB):
            pltpu.make_async_copy(idx_hbm.at[pl.ds(ebase + j * _GSUB, _GSUB)],
                                  idxb.at[slot, j], sem.at[slot]).wait()
        pltpu.make_async_copy(msg_hbm.at[pl.ds(ebase, _G), pl.ds(q * 32, 32)],
                              msgb.at[slot], sem.at[slot]).wait()

    for q in range(4):
        def _zero_stage(i, carry):
            stage[i, pl.ds(0, 16)] = zero16
            stage[i, pl.ds(16, 16)] = zero16
            return carry
        lax.fori_loop(0, _NCH, _zero_stage, 0)

        def _zero_table(m, carry):
            k = s + m * _NTILES

            @pl.when(k < _NCHUNKS)
            def _():
                pltpu.sync_copy(stage, table.at[pl.ds(k * _NCH, _NCH)])
            return carry
        lax.fori_loop(0, (_NCHUNKS + _NTILES - 1) // _NTILES, _zero_table, 0)
        plsc.subcore_barrier()

        _issue(0, 0, q)

        def _chunk(g, carry):
            slot = lax.rem(g, 2)

            @pl.when(g + 1 < _NITER)
            def _():
                _issue(g + 1, 1 - slot, q)
            _drain(g, slot, q)
            for j in range(_NSUB):
                pltpu.sync_copy(msgb.at[slot, pl.ds(j * _GSUB, _GSUB)],
                                table.at[idxb.at[slot, j]], add=True)
            return carry
        lax.fori_loop(0, _NITER, _chunk, 0)
        plsc.subcore_barrier()

        def _writeout(m, carry):
            k = s + m * _NTILES

            @pl.when(k < _NCHUNKS)
            def _():
                pltpu.sync_copy(table.at[pl.ds(k * _NCH, _NCH)], stage)
                pltpu.sync_copy(stage, out_hbm.at[pl.ds(k * _NCH, _NCH),
                                                  pl.ds(q * 32, 32)])
            return carry
        lax.fori_loop(0, (_NCHUNKS + _NTILES - 1) // _NTILES, _writeout, 0)
        plsc.subcore_barrier()


_GG = 40
_GITER = _EPT // _GG


def _sc_gather_one_core(idx_hbm, tabs, outs, idxb, gb, isem, gsem, s):
    def _issue_idx(m, slot):
        pltpu.async_copy(idx_hbm.at[pl.ds(s * _EPT + m * _GG, _GG)],
                         idxb.at[slot], isem.at[slot])

    def _drain_idx(m, slot):
        pltpu.make_async_copy(idx_hbm.at[pl.ds(s * _EPT + m * _GG, _GG)],
                              idxb.at[slot], isem.at[slot]).wait()

    for p in range(2):
        t_hbm = tabs[p]
        out_hbm = outs[p]

        def _issue_gather(slot):
            pltpu.async_copy(t_hbm.at[idxb.at[slot]], gb.at[slot],
                             gsem.at[slot])

        def _drain_gather(slot):
            pltpu.make_async_copy(t_hbm.at[idxb.at[slot]], gb.at[slot],
                                  gsem.at[slot]).wait()

        def _writeout(m, slot):
            pltpu.sync_copy(gb.at[slot],
                            out_hbm.at[pl.ds(s * _EPT + m * _GG, _GG)])

        _issue_idx(0, 0)

        def _body(m, carry):
            slot = lax.rem(m, 2)
            _drain_idx(m, slot)
            _issue_gather(slot)

            @pl.when(m + 1 < _GITER)
            def _():
                _issue_idx(m + 1, 1 - slot)

            @pl.when(m > 0)
            def _():
                _drain_gather(1 - slot)
                _writeout(m - 1, 1 - slot)
            return carry
        lax.fori_loop(0, _GITER, _body, 0)
        _drain_gather((_GITER - 1) % 2)
        _writeout(_GITER - 1, (_GITER - 1) % 2)


def _sc_gather_kernel(tx_hbm, ti_hbm, cols_hbm, rows_hbm,
                      xi_hbm, ii_hbm, xj_hbm, ij_hbm, idxb, gb, isem, gsem):
    c = lax.axis_index("c")
    s = lax.axis_index("s")

    @pl.when(c == 0)
    def _():
        _sc_gather_one_core(cols_hbm, (tx_hbm, ti_hbm), (xi_hbm, ii_hbm),
                            idxb, gb, isem, gsem, s)

    @pl.when(c == 1)
    def _():
        _sc_gather_one_core(rows_hbm, (tx_hbm, ti_hbm), (xj_hbm, ij_hbm),
                            idxb, gb, isem, gsem, s)


def _gather_stage(tx, ti, cols, rows):
    mesh = plsc.VectorSubcoreMesh(core_axis_name="c", subcore_axis_name="s")
    gather = pl.kernel(
        _sc_gather_kernel,
        mesh=mesh,
        compiler_params=pltpu.CompilerParams(use_tc_tiling_on_sc=False),
        out_type=[jax.ShapeDtypeStruct((E, 128), jnp.float32)
                  for _ in range(4)],
        scratch_types=[
            pltpu.VMEM((2, _GG), jnp.int32),
            pltpu.VMEM((2, _GG, 128), jnp.float32),
            pltpu.SemaphoreType.DMA((2,)),
            pltpu.SemaphoreType.DMA((2,)),
        ],
    )
    return gather(tx, ti, cols, rows)


def _sc_scatter_kernel(past_hbm, fut_hbm, cols_hbm, rows_hbm,
                       outp_hbm, outf_hbm, idxb, msgb, stage, table, sem):
    c = lax.axis_index("c")
    s = lax.axis_index("s")

    @pl.when(c == 0)
    def _():
        _sc_scatter_one_core(past_hbm, cols_hbm, outp_hbm,
                             idxb, msgb, stage, table, sem, s)

    @pl.when(c == 1)
    def _():
        _sc_scatter_one_core(fut_hbm, rows_hbm, outf_hbm,
                             idxb, msgb, stage, table, sem, s)


def _scatter_stage(past, fut, cols, rows):
    mesh = plsc.VectorSubcoreMesh(core_axis_name="c", subcore_axis_name="s")
    scatter = pl.kernel(
        _sc_scatter_kernel,
        mesh=mesh,
        compiler_params=pltpu.CompilerParams(use_tc_tiling_on_sc=False),
        out_type=[jax.ShapeDtypeStruct((N, 128), jnp.float32),
                  jax.ShapeDtypeStruct((N, 128), jnp.float32)],
        scratch_types=[
            pltpu.VMEM((2, _NSUB, _GSUB), jnp.int32),
            pltpu.VMEM((2, _G, 32), jnp.float32),
            pltpu.VMEM((_NCH, 32), jnp.float32),
            pltpu.VMEM_SHARED((N, 32), jnp.float32),
            pltpu.SemaphoreType.DMA((2,)),
        ],
    )
    return scatter(past, fut, cols, rows)


def kernel(x, edge_attr, initial_x, att_edge_attr, params, edge_index):
    rows = edge_index[0]
    cols = edge_index[1]
    tx = jnp.pad(x, ((0, 0), (0, 32)))
    ti = jnp.pad(initial_x, ((0, 0), (0, 32)))
    x_i, init_i, x_j, init_j = _gather_stage(tx, ti, cols, rows)

    eu = params["edge_update"]
    fm = params["create_future_msgs"]
    pm = params["create_past_msgs"]
    cb = params["combine_future_past"]

    edge_w = (eu[0][0], eu[0][1], eu[1][0], eu[1][1], eu[2][0], eu[2][1],
              fm[0][0], fm[0][1], fm[1][0], fm[1][1],
              pm[0][0], pm[0][1], pm[1][0], pm[1][1])
    ue, past, fut = _edge_stage(x_i, x_j, init_i, init_j,
                                edge_attr, att_edge_attr, list(edge_w))

    mp, mf = _scatter_stage(past, fut, cols, rows)

    node_w = [cb[0][0][:128], cb[0][0][128:], cb[0][1],
              cb[1][0], cb[1][1], cb[2][0], cb[2][1]]
    updated_nodes = _node_stage(mp, mf, node_w)
    return (updated_nodes, ue)


# speedup vs baseline: 3.6847x; 1.0928x over previous
import functools

import jax
import jax.numpy as jnp
from jax import lax
from jax.experimental import pallas as pl
from jax.experimental.pallas import tpu as pltpu
from jax.experimental.pallas import tpu_sc as plsc

E = 800000
N = 50000
BE = 4000
BN = 2000

_PREC = lax.Precision.DEFAULT

_NTILES = 16
_EPT = E // _NTILES
_G = 200
_GSUB = 40
_NSUB = _G // _GSUB
_NITER = _EPT // _G
_NCH = 250
_NCHUNKS = N // _NCH


def _dot(a, b):
    return lax.dot_general(a, b, (((1,), (0,)), ((), ())),
                           precision=_PREC, preferred_element_type=jnp.float32)


def _edge_block_kernel(xi_ref, xj_ref, ii_ref, ij_ref, ea_ref, aea_ref,
                       w_refs, ue_ref, past_ref, fut_ref):
    (eu_w1, eu_b1, eu_w2, eu_b2, eu_w3, eu_b3,
     f_w1, f_b1, f_w2, f_b2,
     p_w1, p_b1, p_w2, p_b2) = w_refs
    x_i = xi_ref[:, :96]
    x_j = xj_ref[:, :96]
    init_i = ii_ref[:, :96]
    init_j = ij_ref[:, :96]
    ea = ea_ref[...]
    aea = aea_ref[...]

    feats = jnp.concatenate([x_i, x_j, ea, aea], axis=1)
    h = jax.nn.relu(_dot(feats, eu_w1[...]) + eu_b1[...])
    h = jax.nn.relu(_dot(h, eu_w2[...]) + eu_b2[...])
    ue = _dot(h, eu_w3[...]) + eu_b3[...]
    ue_ref[...] = ue

    pfeat = jnp.concatenate([x_j, ue, init_j], axis=1)
    hp = jax.nn.relu(_dot(pfeat, p_w1[...]) + p_b1[...])
    past_ref[...] = _dot(hp, p_w2[...]) + p_b2[...]

    ffeat = jnp.concatenate([x_i, ue, init_i], axis=1)
    hf = jax.nn.relu(_dot(ffeat, f_w1[...]) + f_b1[...])
    fut_ref[...] = _dot(hf, f_w2[...]) + f_b2[...]


def _node_block_kernel(mp_ref, mf_ref, w_refs, out_ref):
    (w1p, w1f, b1, w2, b2, w3, b3) = w_refs
    h = jax.nn.relu(_dot(mp_ref[...], w1p[...]) + _dot(mf_ref[...], w1f[...])
                    + b1[...])
    h = jax.nn.relu(_dot(h, w2[...]) + b2[...])
    out_ref[...] = _dot(h, w3[...]) + b3[...]


def _edge_stage(x_i, x_j, init_i, init_j, edge_attr, att_edge_attr, wflat):
    nblk = E // BE
    eb = lambda i: (i, 0)
    wspec = [pl.BlockSpec(w.shape, lambda i, nd=w.ndim: (0,) * nd) for w in wflat]
    grid_spec = pltpu.PrefetchScalarGridSpec(
        num_scalar_prefetch=0,
        grid=(nblk,),
        in_specs=[
            pl.BlockSpec((BE, 128), eb),
            pl.BlockSpec((BE, 128), eb),
            pl.BlockSpec((BE, 128), eb),
            pl.BlockSpec((BE, 128), eb),
            pl.BlockSpec((BE, 64), eb),
            pl.BlockSpec((BE, 64), eb),
            wspec,
        ],
        out_specs=[
            pl.BlockSpec((BE, 64), eb),
            pl.BlockSpec((BE, 128), eb),
            pl.BlockSpec((BE, 128), eb),
        ],
    )
    return pl.pallas_call(
        _edge_block_kernel,
        grid_spec=grid_spec,
        out_shape=[
            jax.ShapeDtypeStruct((E, 64), jnp.float32),
            jax.ShapeDtypeStruct((E, 128), jnp.float32),
            jax.ShapeDtypeStruct((E, 128), jnp.float32),
        ],
    )(x_i, x_j, init_i, init_j, edge_attr, att_edge_attr, wflat)


def _node_stage(mp, mf, wflat):
    nblk = N // BN
    wspec = [pl.BlockSpec(w.shape, lambda i, nd=w.ndim: (0,) * nd) for w in wflat]
    grid_spec = pltpu.PrefetchScalarGridSpec(
        num_scalar_prefetch=0,
        grid=(nblk,),
        in_specs=[pl.BlockSpec((BN, 128), lambda i: (i, 0)),
                  pl.BlockSpec((BN, 128), lambda i: (i, 0)),
                  wspec],
        out_specs=pl.BlockSpec((BN, 96), lambda i: (i, 0)),
    )
    return pl.pallas_call(
        _node_block_kernel,
        grid_spec=grid_spec,
        out_shape=jax.ShapeDtypeStruct((N, 96), jnp.float32),
    )(mp, mf, wflat)


def _sc_scatter_one_core(msg_hbm, idx_hbm, out_hbm, idxb, msgb, stage, table,
                         sem, s):
    zero16 = jnp.zeros((16,), jnp.float32)

    def _issue(g, slot, q):
        ebase = s * _EPT + g * _G
        for j in range(_NSUB):
            pltpu.async_copy(idx_hbm.at[pl.ds(ebase + j * _GSUB, _GSUB)],
                             idxb.at[slot, j], sem.at[slot])
        pltpu.async_copy(msg_hbm.at[pl.ds(ebase, _G), pl.ds(q * 32, 32)],
                         msgb.at[slot], sem.at[slot])

    def _drain(g, slot, q):
        ebase = s * _EPT + g * _G
        for j in range(_NSUB):
            pltpu.make_async_copy(idx_hbm.at[pl.ds(ebase + j * _GSUB, _GSUB)],
                                  idxb.at[slot, j], sem.at[slot]).wait()
        pltpu.make_async_copy(msg_hbm.at[pl.ds(ebase, _G), pl.ds(q * 32, 32)],
                              msgb.at[slot], sem.at[slot]).wait()

    for q in range(4):
        def _zero_stage(i, carry):
            stage[i, pl.ds(0, 16)] = zero16
            stage[i, pl.ds(16, 16)] = zero16
            return carry
        lax.fori_loop(0, _NCH, _zero_stage, 0)

        def _zero_table(m, carry):
            k = s + m * _NTILES

            @pl.when(k < _NCHUNKS)
            def _():
                pltpu.sync_copy(stage, table.at[pl.ds(k * _NCH, _NCH)])
            return carry
        lax.fori_loop(0, (_NCHUNKS + _NTILES - 1) // _NTILES, _zero_table, 0)
        plsc.subcore_barrier()

        _issue(0, 0, q)

        def _chunk(g, carry):
            slot = lax.rem(g, 2)

            @pl.when(g + 1 < _NITER)
            def _():
                _issue(g + 1, 1 - slot, q)
            _drain(g, slot, q)
            for j in range(_NSUB):
                pltpu.sync_copy(msgb.at[slot, pl.ds(j * _GSUB, _GSUB)],
                                table.at[idxb.at[slot, j]], add=True)
            return carry
        lax.fori_loop(0, _NITER, _chunk, 0)
        plsc.subcore_barrier()

        def _writeout(m, carry):
            k = s + m * _NTILES

            @pl.when(k < _NCHUNKS)
            def _():
                pltpu.sync_copy(table.at[pl.ds(k * _NCH, _NCH)], stage)
                pltpu.sync_copy(stage, out_hbm.at[pl.ds(k * _NCH, _NCH),
                                                  pl.ds(q * 32, 32)])
            return carry
        lax.fori_loop(0, (_NCHUNKS + _NTILES - 1) // _NTILES, _writeout, 0)
        plsc.subcore_barrier()


_GG = 40
_GITER = _EPT // _GG


def _sc_gather_one_core(idx_hbm, tabs, outs, idxb, gb1, gb2,
                        isem, gsem, wsem, s):
    t1, t2 = tabs
    o1, o2 = outs

    def _issue_idx(m, slot):
        pltpu.async_copy(idx_hbm.at[pl.ds(s * _EPT + m * _GG, _GG)],
                         idxb.at[slot], isem.at[slot])

    def _drain_idx(m, slot):
        pltpu.make_async_copy(idx_hbm.at[pl.ds(s * _EPT + m * _GG, _GG)],
                              idxb.at[slot], isem.at[slot]).wait()

    def _issue_g(slot):
        pltpu.async_copy(t1.at[idxb.at[slot]], gb1.at[slot], gsem.at[slot])
        pltpu.async_copy(t2.at[idxb.at[slot]], gb2.at[slot], gsem.at[slot])

    def _drain_g(slot):
        pltpu.make_async_copy(t1.at[idxb.at[slot]], gb1.at[slot],
                              gsem.at[slot]).wait()
        pltpu.make_async_copy(t2.at[idxb.at[slot]], gb2.at[slot],
                              gsem.at[slot]).wait()

    def _issue_w(m, slot):
        dst = pl.ds(s * _EPT + m * _GG, _GG)
        pltpu.async_copy(gb1.at[slot], o1.at[dst], wsem.at[slot])
        pltpu.async_copy(gb2.at[slot], o2.at[dst], wsem.at[slot])

    def _drain_w(m, slot):
        dst = pl.ds(s * _EPT + m * _GG, _GG)
        pltpu.make_async_copy(gb1.at[slot], o1.at[dst], wsem.at[slot]).wait()
        pltpu.make_async_copy(gb2.at[slot], o2.at[dst], wsem.at[slot]).wait()

    _issue_idx(0, 0)

    def _body(m, carry):
        slot = lax.rem(m, 2)
        _drain_idx(m, slot)

        @pl.when(m >= 2)
        def _():
            _drain_w(m - 2, slot)
        _issue_g(slot)

        @pl.when(m > 0)
        def _():
            _drain_g(1 - slot)
            _issue_w(m - 1, 1 - slot)

        @pl.when(m + 1 < _GITER)
        def _():
            _issue_idx(m + 1, 1 - slot)
        return carry
    lax.fori_loop(0, _GITER, _body, 0)

    last = (_GITER - 1) % 2
    _drain_g(last)
    _issue_w(_GITER - 1, last)
    _drain_w(_GITER - 2, 1 - last)
    _drain_w(_GITER - 1, last)


def _sc_gather_kernel(tx_hbm, ti_hbm, cols_hbm, rows_hbm,
                      xi_hbm, ii_hbm, xj_hbm, ij_hbm,
                      idxb, gb1, gb2, isem, gsem, wsem):
    c = lax.axis_index("c")
    s = lax.axis_index("s")

    @pl.when(c == 0)
    def _():
        _sc_gather_one_core(cols_hbm, (tx_hbm, ti_hbm), (xi_hbm, ii_hbm),
                            idxb, gb1, gb2, isem, gsem, wsem, s)

    @pl.when(c == 1)
    def _():
        _sc_gather_one_core(rows_hbm, (tx_hbm, ti_hbm), (xj_hbm, ij_hbm),
                            idxb, gb1, gb2, isem, gsem, wsem, s)


def _gather_stage(tx, ti, cols, rows):
    mesh = plsc.VectorSubcoreMesh(core_axis_name="c", subcore_axis_name="s")
    gather = pl.kernel(
        _sc_gather_kernel,
        mesh=mesh,
        compiler_params=pltpu.CompilerParams(use_tc_tiling_on_sc=False),
        out_type=[jax.ShapeDtypeStruct((E, 128), jnp.float32)
                  for _ in range(4)],
        scratch_types=[
            pltpu.VMEM((2, _GG), jnp.int32),
            pltpu.VMEM((2, _GG, 128), jnp.float32),
            pltpu.VMEM((2, _GG, 128), jnp.float32),
            pltpu.SemaphoreType.DMA((2,)),
            pltpu.SemaphoreType.DMA((2,)),
            pltpu.SemaphoreType.DMA((2,)),
        ],
    )
    return gather(tx, ti, cols, rows)


def _sc_scatter_kernel(past_hbm, fut_hbm, cols_hbm, rows_hbm,
                       outp_hbm, outf_hbm, idxb, msgb, stage, table, sem):
    c = lax.axis_index("c")
    s = lax.axis_index("s")

    @pl.when(c == 0)
    def _():
        _sc_scatter_one_core(past_hbm, cols_hbm, outp_hbm,
                             idxb, msgb, stage, table, sem, s)

    @pl.when(c == 1)
    def _():
        _sc_scatter_one_core(fut_hbm, rows_hbm, outf_hbm,
                             idxb, msgb, stage, table, sem, s)


def _scatter_stage(past, fut, cols, rows):
    mesh = plsc.VectorSubcoreMesh(core_axis_name="c", subcore_axis_name="s")
    scatter = pl.kernel(
        _sc_scatter_kernel,
        mesh=mesh,
        compiler_params=pltpu.CompilerParams(use_tc_tiling_on_sc=False),
        out_type=[jax.ShapeDtypeStruct((N, 128), jnp.float32),
                  jax.ShapeDtypeStruct((N, 128), jnp.float32)],
        scratch_types=[
            pltpu.VMEM((2, _NSUB, _GSUB), jnp.int32),
            pltpu.VMEM((2, _G, 32), jnp.float32),
            pltpu.VMEM((_NCH, 32), jnp.float32),
            pltpu.VMEM_SHARED((N, 32), jnp.float32),
            pltpu.SemaphoreType.DMA((2,)),
        ],
    )
    return scatter(past, fut, cols, rows)


def kernel(x, edge_attr, initial_x, att_edge_attr, params, edge_index):
    rows = edge_index[0]
    cols = edge_index[1]
    tx = jnp.pad(x, ((0, 0), (0, 32)))
    ti = jnp.pad(initial_x, ((0, 0), (0, 32)))
    x_i, init_i, x_j, init_j = _gather_stage(tx, ti, cols, rows)

    eu = params["edge_update"]
    fm = params["create_future_msgs"]
    pm = params["create_past_msgs"]
    cb = params["combine_future_past"]

    edge_w = (eu[0][0], eu[0][1], eu[1][0], eu[1][1], eu[2][0], eu[2][1],
              fm[0][0], fm[0][1], fm[1][0], fm[1][1],
              pm[0][0], pm[0][1], pm[1][0], pm[1][1])
    ue, past, fut = _edge_stage(x_i, x_j, init_i, init_j,
                                edge_attr, att_edge_attr, list(edge_w))

    mp, mf = _scatter_stage(past, fut, cols, rows)

    node_w = [cb[0][0][:128], cb[0][0][128:], cb[0][1],
              cb[1][0], cb[1][1], cb[2][0], cb[2][1]]
    updated_nodes = _node_stage(mp, mf, node_w)
    return (updated_nodes, ue)


# speedup vs baseline: 3.9725x; 1.0781x over previous
import functools

import jax
import jax.numpy as jnp
from jax import lax
from jax.experimental import pallas as pl
from jax.experimental.pallas import tpu as pltpu
from jax.experimental.pallas import tpu_sc as plsc

E = 800000
N = 50000
BE = 4000
BN = 2000

_PREC = lax.Precision.DEFAULT

_NTILES = 16
_EPT = E // _NTILES
_G = 400
_GSUB = 80
_NSUB = _G // _GSUB
_NITER = _EPT // _G
_NCH = 125
_NCHUNKS = N // _NCH


def _dot(a, b):
    return lax.dot_general(a, b, (((1,), (0,)), ((), ())),
                           precision=_PREC, preferred_element_type=jnp.float32)


def _edge_block_kernel(xi_ref, xj_ref, ii_ref, ij_ref, ea_ref, aea_ref,
                       w_refs, ue_ref, past_ref, fut_ref):
    (eu_w1, eu_b1, eu_w2, eu_b2, eu_w3, eu_b3,
     f_w1, f_b1, f_w2, f_b2,
     p_w1, p_b1, p_w2, p_b2) = w_refs
    x_i = xi_ref[:, :96]
    x_j = xj_ref[:, :96]
    init_i = ii_ref[:, :96]
    init_j = ij_ref[:, :96]
    ea = ea_ref[...]
    aea = aea_ref[...]

    feats = jnp.concatenate([x_i, x_j, ea, aea], axis=1)
    h = jax.nn.relu(_dot(feats, eu_w1[...]) + eu_b1[...])
    h = jax.nn.relu(_dot(h, eu_w2[...]) + eu_b2[...])
    ue = _dot(h, eu_w3[...]) + eu_b3[...]
    ue_ref[...] = ue

    pfeat = jnp.concatenate([x_j, ue, init_j], axis=1)
    hp = jax.nn.relu(_dot(pfeat, p_w1[...]) + p_b1[...])
    past_ref[...] = _dot(hp, p_w2[...]) + p_b2[...]

    ffeat = jnp.concatenate([x_i, ue, init_i], axis=1)
    hf = jax.nn.relu(_dot(ffeat, f_w1[...]) + f_b1[...])
    fut_ref[...] = _dot(hf, f_w2[...]) + f_b2[...]


def _node_block_kernel(mp_ref, mf_ref, w_refs, out_ref):
    (w1p, w1f, b1, w2, b2, w3, b3) = w_refs
    h = jax.nn.relu(_dot(mp_ref[...], w1p[...]) + _dot(mf_ref[...], w1f[...])
                    + b1[...])
    h = jax.nn.relu(_dot(h, w2[...]) + b2[...])
    out_ref[...] = _dot(h, w3[...]) + b3[...]


def _edge_stage(x_i, x_j, init_i, init_j, edge_attr, att_edge_attr, wflat):
    nblk = E // BE
    eb = lambda i: (i, 0)
    wspec = [pl.BlockSpec(w.shape, lambda i, nd=w.ndim: (0,) * nd) for w in wflat]
    grid_spec = pltpu.PrefetchScalarGridSpec(
        num_scalar_prefetch=0,
        grid=(nblk,),
        in_specs=[
            pl.BlockSpec((BE, 128), eb),
            pl.BlockSpec((BE, 128), eb),
            pl.BlockSpec((BE, 128), eb),
            pl.BlockSpec((BE, 128), eb),
            pl.BlockSpec((BE, 64), eb),
            pl.BlockSpec((BE, 64), eb),
            wspec,
        ],
        out_specs=[
            pl.BlockSpec((BE, 64), eb),
            pl.BlockSpec((BE, 128), eb),
            pl.BlockSpec((BE, 128), eb),
        ],
    )
    return pl.pallas_call(
        _edge_block_kernel,
        grid_spec=grid_spec,
        out_shape=[
            jax.ShapeDtypeStruct((E, 64), jnp.float32),
            jax.ShapeDtypeStruct((E, 128), jnp.float32),
            jax.ShapeDtypeStruct((E, 128), jnp.float32),
        ],
    )(x_i, x_j, init_i, init_j, edge_attr, att_edge_attr, wflat)


def _node_stage(mp, mf, wflat):
    nblk = N // BN
    wspec = [pl.BlockSpec(w.shape, lambda i, nd=w.ndim: (0,) * nd) for w in wflat]
    grid_spec = pltpu.PrefetchScalarGridSpec(
        num_scalar_prefetch=0,
        grid=(nblk,),
        in_specs=[pl.BlockSpec((BN, 128), lambda i: (i, 0)),
                  pl.BlockSpec((BN, 128), lambda i: (i, 0)),
                  wspec],
        out_specs=pl.BlockSpec((BN, 96), lambda i: (i, 0)),
    )
    return pl.pallas_call(
        _node_block_kernel,
        grid_spec=grid_spec,
        out_shape=jax.ShapeDtypeStruct((N, 96), jnp.float32),
    )(mp, mf, wflat)


def _sc_scatter_one_core(msg_hbm, idx_hbm, out_hbm, idxb, msgb, stage, table,
                         sem, s):
    zero16 = jnp.zeros((16,), jnp.float32)

    def _issue(g, slot, q):
        ebase = s * _EPT + g * _G
        for j in range(_NSUB):
            pltpu.async_copy(idx_hbm.at[pl.ds(ebase + j * _GSUB, _GSUB)],
                             idxb.at[slot, j], sem.at[slot])
        pltpu.async_copy(msg_hbm.at[pl.ds(ebase, _G), pl.ds(q * 32, 32)],
                         msgb.at[slot], sem.at[slot])

    def _drain(g, slot, q):
        ebase = s * _EPT + g * _G
        for j in range(_NSUB):
            pltpu.make_async_copy(idx_hbm.at[pl.ds(ebase + j * _GSUB, _GSUB)],
                                  idxb.at[slot, j], sem.at[slot]).wait()
        pltpu.make_async_copy(msg_hbm.at[pl.ds(ebase, _G), pl.ds(q * 32, 32)],
                              msgb.at[slot], sem.at[slot]).wait()

    for q in range(4):
        def _zero_stage(i, carry):
            stage[i, pl.ds(0, 16)] = zero16
            stage[i, pl.ds(16, 16)] = zero16
            return carry
        lax.fori_loop(0, _NCH, _zero_stage, 0)

        def _zero_table(m, carry):
            k = s + m * _NTILES

            @pl.when(k < _NCHUNKS)
            def _():
                pltpu.sync_copy(stage, table.at[pl.ds(k * _NCH, _NCH)])
            return carry
        lax.fori_loop(0, (_NCHUNKS + _NTILES - 1) // _NTILES, _zero_table, 0)
        plsc.subcore_barrier()

        _issue(0, 0, q)

        def _chunk(g, carry):
            slot = lax.rem(g, 2)

            @pl.when(g + 1 < _NITER)
            def _():
                _issue(g + 1, 1 - slot, q)
            _drain(g, slot, q)
            for j in range(_NSUB):
                pltpu.sync_copy(msgb.at[slot, pl.ds(j * _GSUB, _GSUB)],
                                table.at[idxb.at[slot, j]], add=True)
            return carry
        lax.fori_loop(0, _NITER, _chunk, 0)
        plsc.subcore_barrier()

        def _writeout(m, carry):
            k = s + m * _NTILES

            @pl.when(k < _NCHUNKS)
            def _():
                pltpu.sync_copy(table.at[pl.ds(k * _NCH, _NCH)], stage)
                pltpu.sync_copy(stage, out_hbm.at[pl.ds(k * _NCH, _NCH),
                                                  pl.ds(q * 32, 32)])
            return carry
        lax.fori_loop(0, (_NCHUNKS + _NTILES - 1) // _NTILES, _writeout, 0)
        plsc.subcore_barrier()


def _pad_block_kernel(x_ref, i_ref, tx_ref, ti_ref):
    z = jnp.zeros((BN, 32), jnp.float32)
    tx_ref[...] = jnp.concatenate([x_ref[...], z], axis=1)
    ti_ref[...] = jnp.concatenate([i_ref[...], z], axis=1)


def _pad_stage(x, initial_x):
    nblk = N // BN
    grid_spec = pltpu.PrefetchScalarGridSpec(
        num_scalar_prefetch=0,
        grid=(nblk,),
        in_specs=[pl.BlockSpec((BN, 96), lambda i: (i, 0)),
                  pl.BlockSpec((BN, 96), lambda i: (i, 0))],
        out_specs=[pl.BlockSpec((BN, 128), lambda i: (i, 0)),
                   pl.BlockSpec((BN, 128), lambda i: (i, 0))],
    )
    return pl.pallas_call(
        _pad_block_kernel,
        grid_spec=grid_spec,
        out_shape=[jax.ShapeDtypeStruct((N, 128), jnp.float32),
                   jax.ShapeDtypeStruct((N, 128), jnp.float32)],
    )(x, initial_x)


_GG = 40
_GITER = _EPT // _GG


def _sc_gather_one_core(idx_hbm, tabs, outs, idxb, gb1, gb2,
                        isem, gsem, wsem, s):
    t1, t2 = tabs
    o1, o2 = outs

    def _issue_idx(m, slot):
        pltpu.async_copy(idx_hbm.at[pl.ds(s * _EPT + m * _GG, _GG)],
                         idxb.at[slot], isem.at[slot])

    def _drain_idx(m, slot):
        pltpu.make_async_copy(idx_hbm.at[pl.ds(s * _EPT + m * _GG, _GG)],
                              idxb.at[slot], isem.at[slot]).wait()

    def _issue_g(slot):
        pltpu.async_copy(t1.at[idxb.at[slot]], gb1.at[slot], gsem.at[slot])
        pltpu.async_copy(t2.at[idxb.at[slot]], gb2.at[slot], gsem.at[slot])

    def _drain_g(slot):
        pltpu.make_async_copy(t1.at[idxb.at[slot]], gb1.at[slot],
                              gsem.at[slot]).wait()
        pltpu.make_async_copy(t2.at[idxb.at[slot]], gb2.at[slot],
                              gsem.at[slot]).wait()

    def _issue_w(m, slot):
        dst = pl.ds(s * _EPT + m * _GG, _GG)
        pltpu.async_copy(gb1.at[slot], o1.at[dst], wsem.at[slot])
        pltpu.async_copy(gb2.at[slot], o2.at[dst], wsem.at[slot])

    def _drain_w(m, slot):
        dst = pl.ds(s * _EPT + m * _GG, _GG)
        pltpu.make_async_copy(gb1.at[slot], o1.at[dst], wsem.at[slot]).wait()
        pltpu.make_async_copy(gb2.at[slot], o2.at[dst], wsem.at[slot]).wait()

    _issue_idx(0, 0)

    def _body(m, carry):
        slot = lax.rem(m, 2)
        _drain_idx(m, slot)

        @pl.when(m >= 2)
        def _():
            _drain_w(m - 2, slot)
        _issue_g(slot)

        @pl.when(m > 0)
        def _():
            _drain_g(1 - slot)
            _issue_w(m - 1, 1 - slot)

        @pl.when(m + 1 < _GITER)
        def _():
            _issue_idx(m + 1, 1 - slot)
        return carry
    lax.fori_loop(0, _GITER, _body, 0)

    last = (_GITER - 1) % 2
    _drain_g(last)
    _issue_w(_GITER - 1, last)
    _drain_w(_GITER - 2, 1 - last)
    _drain_w(_GITER - 1, last)


def _sc_gather_kernel(tx_hbm, ti_hbm, cols_hbm, rows_hbm,
                      xi_hbm, ii_hbm, xj_hbm, ij_hbm,
                      idxb, gb1, gb2, isem, gsem, wsem):
    c = lax.axis_index("c")
    s = lax.axis_index("s")

    @pl.when(c == 0)
    def _():
        _sc_gather_one_core(cols_hbm, (tx_hbm, ti_hbm), (xi_hbm, ii_hbm),
                            idxb, gb1, gb2, isem, gsem, wsem, s)

    @pl.when(c == 1)
    def _():
        _sc_gather_one_core(rows_hbm, (tx_hbm, ti_hbm), (xj_hbm, ij_hbm),
                            idxb, gb1, gb2, isem, gsem, wsem, s)


def _gather_stage(tx, ti, cols, rows):
    mesh = plsc.VectorSubcoreMesh(core_axis_name="c", subcore_axis_name="s")
    gather = pl.kernel(
        _sc_gather_kernel,
        mesh=mesh,
        compiler_params=pltpu.CompilerParams(use_tc_tiling_on_sc=False),
        out_type=[jax.ShapeDtypeStruct((E, 128), jnp.float32)
                  for _ in range(4)],
        scratch_types=[
            pltpu.VMEM((2, _GG), jnp.int32),
            pltpu.VMEM((2, _GG, 128), jnp.float32),
            pltpu.VMEM((2, _GG, 128), jnp.float32),
            pltpu.SemaphoreType.DMA((2,)),
            pltpu.SemaphoreType.DMA((2,)),
            pltpu.SemaphoreType.DMA((2,)),
        ],
    )
    return gather(tx, ti, cols, rows)


def _sc_scatter_kernel(past_hbm, fut_hbm, cols_hbm, rows_hbm,
                       outp_hbm, outf_hbm, idxb, msgb, stage, table, sem):
    c = lax.axis_index("c")
    s = lax.axis_index("s")

    @pl.when(c == 0)
    def _():
        _sc_scatter_one_core(past_hbm, cols_hbm, outp_hbm,
                             idxb, msgb, stage, table, sem, s)

    @pl.when(c == 1)
    def _():
        _sc_scatter_one_core(fut_hbm, rows_hbm, outf_hbm,
                             idxb, msgb, stage, table, sem, s)


def _scatter_stage(past, fut, cols, rows):
    mesh = plsc.VectorSubcoreMesh(core_axis_name="c", subcore_axis_name="s")
    scatter = pl.kernel(
        _sc_scatter_kernel,
        mesh=mesh,
        compiler_params=pltpu.CompilerParams(use_tc_tiling_on_sc=False),
        out_type=[jax.ShapeDtypeStruct((N, 128), jnp.float32),
                  jax.ShapeDtypeStruct((N, 128), jnp.float32)],
        scratch_types=[
            pltpu.VMEM((2, _NSUB, _GSUB), jnp.int32),
            pltpu.VMEM((2, _G, 32), jnp.float32),
            pltpu.VMEM((_NCH, 32), jnp.float32),
            pltpu.VMEM_SHARED((N, 32), jnp.float32),
            pltpu.SemaphoreType.DMA((2,)),
        ],
    )
    return scatter(past, fut, cols, rows)


def kernel(x, edge_attr, initial_x, att_edge_attr, params, edge_index):
    rows = edge_index[0]
    cols = edge_index[1]
    tx, ti = _pad_stage(x, initial_x)
    x_i, init_i, x_j, init_j = _gather_stage(tx, ti, cols, rows)

    eu = params["edge_update"]
    fm = params["create_future_msgs"]
    pm = params["create_past_msgs"]
    cb = params["combine_future_past"]

    edge_w = (eu[0][0], eu[0][1], eu[1][0], eu[1][1], eu[2][0], eu[2][1],
              fm[0][0], fm[0][1], fm[1][0], fm[1][1],
              pm[0][0], pm[0][1], pm[1][0], pm[1][1])
    ue, past, fut = _edge_stage(x_i, x_j, init_i, init_j,
                                edge_attr, att_edge_attr, list(edge_w))

    mp, mf = _scatter_stage(past, fut, cols, rows)

    node_w = [cb[0][0][:128], cb[0][0][128:], cb[0][1],
              cb[1][0], cb[1][1], cb[2][0], cb[2][1]]
    updated_nodes = _node_stage(mp, mf, node_w)
    return (updated_nodes, ue)


# speedup vs baseline: 4.0707x; 1.0247x over previous
import functools

import jax
import jax.numpy as jnp
from jax import lax
from jax.experimental import pallas as pl
from jax.experimental.pallas import tpu as pltpu
from jax.experimental.pallas import tpu_sc as plsc

E = 800000
N = 50000
BE = 4000
BN = 2000

_PREC = lax.Precision.DEFAULT

_NTILES = 16
_EPT = E // _NTILES
_G = 400
_GSUB = 80
_NSUB = _G // _GSUB
_NITER = _EPT // _G
_NCH = 125
_NCHUNKS = N // _NCH


def _dot(a, b):
    return lax.dot_general(a, b, (((1,), (0,)), ((), ())),
                           precision=_PREC, preferred_element_type=jnp.float32)


def _edge_block_kernel(xi_ref, xj_ref, ii_ref, ij_ref, ea_ref, aea_ref,
                       w_refs, ue_ref, past_ref, fut_ref):
    (eu_w1, eu_b1, eu_w2, eu_b2, eu_w3, eu_b3,
     f_w1, f_b1, f_w2, f_b2,
     p_w1, p_b1, p_w2, p_b2) = w_refs
    x_i = xi_ref[:, :96]
    x_j = xj_ref[:, :96]
    init_i = ii_ref[:, :96]
    init_j = ij_ref[:, :96]
    ea = ea_ref[...]
    aea = aea_ref[...]

    feats = jnp.concatenate([x_i, x_j, ea, aea], axis=1)
    h = jax.nn.relu(_dot(feats, eu_w1[...]) + eu_b1[...])
    h = jax.nn.relu(_dot(h, eu_w2[...]) + eu_b2[...])
    ue = _dot(h, eu_w3[...]) + eu_b3[...]
    ue_ref[...] = ue

    pfeat = jnp.concatenate([x_j, ue, init_j], axis=1)
    hp = jax.nn.relu(_dot(pfeat, p_w1[...]) + p_b1[...])
    past_ref[...] = _dot(hp, p_w2[...]) + p_b2[...]

    ffeat = jnp.concatenate([x_i, ue, init_i], axis=1)
    hf = jax.nn.relu(_dot(ffeat, f_w1[...]) + f_b1[...])
    fut_ref[...] = _dot(hf, f_w2[...]) + f_b2[...]


def _node_block_kernel(mp_ref, mf_ref, w_refs, out_ref):
    (w1p, w1f, b1, w2, b2, w3, b3) = w_refs
    h = jax.nn.relu(_dot(mp_ref[...], w1p[...]) + _dot(mf_ref[...], w1f[...])
                    + b1[...])
    h = jax.nn.relu(_dot(h, w2[...]) + b2[...])
    out_ref[...] = _dot(h, w3[...]) + b3[...]


def _edge_stage(x_i, x_j, init_i, init_j, edge_attr, att_edge_attr, wflat):
    nblk = E // BE
    eb = lambda i: (i, 0)
    wspec = [pl.BlockSpec(w.shape, lambda i, nd=w.ndim: (0,) * nd) for w in wflat]
    grid_spec = pltpu.PrefetchScalarGridSpec(
        num_scalar_prefetch=0,
        grid=(nblk,),
        in_specs=[
            pl.BlockSpec((BE, 128), eb),
            pl.BlockSpec((BE, 128), eb),
            pl.BlockSpec((BE, 128), eb),
            pl.BlockSpec((BE, 128), eb),
            pl.BlockSpec((BE, 64), eb),
            pl.BlockSpec((BE, 64), eb),
            wspec,
        ],
        out_specs=[
            pl.BlockSpec((BE, 64), eb),
            pl.BlockSpec((BE, 128), eb),
            pl.BlockSpec((BE, 128), eb),
        ],
    )
    return pl.pallas_call(
        _edge_block_kernel,
        grid_spec=grid_spec,
        out_shape=[
            jax.ShapeDtypeStruct((E, 64), jnp.float32),
            jax.ShapeDtypeStruct((E, 128), jnp.float32),
            jax.ShapeDtypeStruct((E, 128), jnp.float32),
        ],
    )(x_i, x_j, init_i, init_j, edge_attr, att_edge_attr, wflat)


def _node_stage(mp, mf, wflat):
    nblk = N // BN
    wspec = [pl.BlockSpec(w.shape, lambda i, nd=w.ndim: (0,) * nd) for w in wflat]
    grid_spec = pltpu.PrefetchScalarGridSpec(
        num_scalar_prefetch=0,
        grid=(nblk,),
        in_specs=[pl.BlockSpec((BN, 128), lambda i: (i, 0)),
                  pl.BlockSpec((BN, 128), lambda i: (i, 0)),
                  wspec],
        out_specs=pl.BlockSpec((BN, 96), lambda i: (i, 0)),
    )
    return pl.pallas_call(
        _node_block_kernel,
        grid_spec=grid_spec,
        out_shape=jax.ShapeDtypeStruct((N, 96), jnp.float32),
    )(mp, mf, wflat)


def _sc_scatter_one_core(msg_hbm, idx_hbm, out_hbm, idxb, msgb, stage, table,
                         sem, s):
    zero16 = jnp.zeros((16,), jnp.float32)

    def _issue(g, slot, q):
        ebase = s * _EPT + g * _G
        for j in range(_NSUB):
            pltpu.async_copy(idx_hbm.at[pl.ds(ebase + j * _GSUB, _GSUB)],
                             idxb.at[slot, j], sem.at[slot])
        pltpu.async_copy(msg_hbm.at[pl.ds(ebase, _G), pl.ds(q * 32, 32)],
                         msgb.at[slot], sem.at[slot])

    def _drain(g, slot, q):
        ebase = s * _EPT + g * _G
        for j in range(_NSUB):
            pltpu.make_async_copy(idx_hbm.at[pl.ds(ebase + j * _GSUB, _GSUB)],
                                  idxb.at[slot, j], sem.at[slot]).wait()
        pltpu.make_async_copy(msg_hbm.at[pl.ds(ebase, _G), pl.ds(q * 32, 32)],
                              msgb.at[slot], sem.at[slot]).wait()

    for q in range(4):
        def _zero_stage(i, carry):
            stage[i, pl.ds(0, 16)] = zero16
            stage[i, pl.ds(16, 16)] = zero16
            return carry
        lax.fori_loop(0, _NCH, _zero_stage, 0)

        def _zero_table(m, carry):
            k = s + m * _NTILES

            @pl.when(k < _NCHUNKS)
            def _():
                pltpu.sync_copy(stage, table.at[pl.ds(k * _NCH, _NCH)])
            return carry
        lax.fori_loop(0, (_NCHUNKS + _NTILES - 1) // _NTILES, _zero_table, 0)
        plsc.subcore_barrier()

        _issue(0, 0, q)

        def _chunk(g, carry):
            slot = lax.rem(g, 2)

            @pl.when(g + 1 < _NITER)
            def _():
                _issue(g + 1, 1 - slot, q)
            _drain(g, slot, q)
            for j in range(_NSUB):
                pltpu.sync_copy(msgb.at[slot, pl.ds(j * _GSUB, _GSUB)],
                                table.at[idxb.at[slot, j]], add=True)
            return carry
        lax.fori_loop(0, _NITER, _chunk, 0)
        plsc.subcore_barrier()

        def _writeout(m, carry):
            k = s + m * _NTILES

            @pl.when(k < _NCHUNKS)
            def _():
                pltpu.sync_copy(table.at[pl.ds(k * _NCH, _NCH)], stage)
                pltpu.sync_copy(stage, out_hbm.at[pl.ds(k * _NCH, _NCH),
                                                  pl.ds(q * 32, 32)])
            return carry
        lax.fori_loop(0, (_NCHUNKS + _NTILES - 1) // _NTILES, _writeout, 0)
        plsc.subcore_barrier()


def _pad_block_kernel(x_ref, i_ref, tx_ref, ti_ref):
    z = jnp.zeros((BN, 32), jnp.float32)
    tx_ref[...] = jnp.concatenate([x_ref[...], z], axis=1)
    ti_ref[...] = jnp.concatenate([i_ref[...], z], axis=1)


def _pad_stage(x, initial_x):
    nblk = N // BN
    grid_spec = pltpu.PrefetchScalarGridSpec(
        num_scalar_prefetch=0,
        grid=(nblk,),
        in_specs=[pl.BlockSpec((BN, 96), lambda i: (i, 0)),
                  pl.BlockSpec((BN, 96), lambda i: (i, 0))],
        out_specs=[pl.BlockSpec((BN, 128), lambda i: (i, 0)),
                   pl.BlockSpec((BN, 128), lambda i: (i, 0))],
    )
    return pl.pallas_call(
        _pad_block_kernel,
        grid_spec=grid_spec,
        out_shape=[jax.ShapeDtypeStruct((N, 128), jnp.float32),
                   jax.ShapeDtypeStruct((N, 128), jnp.float32)],
    )(x, initial_x)


_GG = 128
_NGCH = E // _GG


def _sc_gather_one_core(idx_hbm, tabs, outs, idxb, gb1, gb2,
                        isem, gsem, wsem, s):
    t1, t2 = tabs
    o1, o2 = outs
    niter = (_NGCH - s + _NTILES - 1) // _NTILES

    def _ebase(m):
        return (s + m * _NTILES) * _GG

    def _issue_idx(m, slot):
        pltpu.async_copy(idx_hbm.at[pl.ds(_ebase(m), _GG)],
                         idxb.at[slot], isem.at[slot])

    def _drain_idx(m, slot):
        pltpu.make_async_copy(idx_hbm.at[pl.ds(_ebase(m), _GG)],
                              idxb.at[slot], isem.at[slot]).wait()

    def _issue_g(slot):
        pltpu.async_copy(t1.at[idxb.at[slot]], gb1.at[slot], gsem.at[slot])
        pltpu.async_copy(t2.at[idxb.at[slot]], gb2.at[slot], gsem.at[slot])

    def _drain_g(slot):
        pltpu.make_async_copy(t1.at[idxb.at[slot]], gb1.at[slot],
                              gsem.at[slot]).wait()
        pltpu.make_async_copy(t2.at[idxb.at[slot]], gb2.at[slot],
                              gsem.at[slot]).wait()

    def _issue_w(m, slot):
        dst = pl.ds(_ebase(m), _GG)
        pltpu.async_copy(gb1.at[slot], o1.at[dst], wsem.at[slot])
        pltpu.async_copy(gb2.at[slot], o2.at[dst], wsem.at[slot])

    def _drain_w(m, slot):
        dst = pl.ds(_ebase(m), _GG)
        pltpu.make_async_copy(gb1.at[slot], o1.at[dst], wsem.at[slot]).wait()
        pltpu.make_async_copy(gb2.at[slot], o2.at[dst], wsem.at[slot]).wait()

    _issue_idx(0, 0)

    def _body(m, carry):
        slot = lax.rem(m, 2)
        _drain_idx(m, slot)

        @pl.when(m >= 2)
        def _():
            _drain_w(m - 2, slot)
        _issue_g(slot)

        @pl.when(m > 0)
        def _():
            _drain_g(1 - slot)
            _issue_w(m - 1, 1 - slot)

        @pl.when(m + 1 < niter)
        def _():
            _issue_idx(m + 1, 1 - slot)
        return carry
    lax.fori_loop(0, niter, _body, 0)

    last = lax.rem(niter - 1, 2)
    _drain_g(last)
    _issue_w(niter - 1, last)
    _drain_w(niter - 2, 1 - last)
    _drain_w(niter - 1, last)


def _sc_gather_kernel(tx_hbm, ti_hbm, cols_hbm, rows_hbm,
                      xi_hbm, ii_hbm, xj_hbm, ij_hbm,
                      idxb, gb1, gb2, isem, gsem, wsem):
    c = lax.axis_index("c")
    s = lax.axis_index("s")

    @pl.when(c == 0)
    def _():
        _sc_gather_one_core(cols_hbm, (tx_hbm, ti_hbm), (xi_hbm, ii_hbm),
                            idxb, gb1, gb2, isem, gsem, wsem, s)

    @pl.when(c == 1)
    def _():
        _sc_gather_one_core(rows_hbm, (tx_hbm, ti_hbm), (xj_hbm, ij_hbm),
                            idxb, gb1, gb2, isem, gsem, wsem, s)


def _gather_stage(tx, ti, cols, rows):
    mesh = plsc.VectorSubcoreMesh(core_axis_name="c", subcore_axis_name="s")
    gather = pl.kernel(
        _sc_gather_kernel,
        mesh=mesh,
        compiler_params=pltpu.CompilerParams(use_tc_tiling_on_sc=False),
        out_type=[jax.ShapeDtypeStruct((E, 128), jnp.float32)
                  for _ in range(4)],
        scratch_types=[
            pltpu.VMEM((2, _GG), jnp.int32),
            pltpu.VMEM((2, _GG, 128), jnp.float32),
            pltpu.VMEM((2, _GG, 128), jnp.float32),

            pltpu.SemaphoreType.DMA((2,)),
            pltpu.SemaphoreType.DMA((2,)),
            pltpu.SemaphoreType.DMA((2,)),
        ],
    )
    return gather(tx, ti, cols, rows)


def _sc_scatter_kernel(past_hbm, fut_hbm, cols_hbm, rows_hbm,
                       outp_hbm, outf_hbm, idxb, msgb, stage, table, sem):
    c = lax.axis_index("c")
    s = lax.axis_index("s")

    @pl.when(c == 0)
    def _():
        _sc_scatter_one_core(past_hbm, cols_hbm, outp_hbm,
                             idxb, msgb, stage, table, sem, s)

    @pl.when(c == 1)
    def _():
        _sc_scatter_one_core(fut_hbm, rows_hbm, outf_hbm,
                             idxb, msgb, stage, table, sem, s)


def _scatter_stage(past, fut, cols, rows):
    mesh = plsc.VectorSubcoreMesh(core_axis_name="c", subcore_axis_name="s")
    scatter = pl.kernel(
        _sc_scatter_kernel,
        mesh=mesh,
        compiler_params=pltpu.CompilerParams(use_tc_tiling_on_sc=False),
        out_type=[jax.ShapeDtypeStruct((N, 128), jnp.float32),
                  jax.ShapeDtypeStruct((N, 128), jnp.float32)],
        scratch_types=[
            pltpu.VMEM((2, _NSUB, _GSUB), jnp.int32),
            pltpu.VMEM((2, _G, 32), jnp.float32),
            pltpu.VMEM((_NCH, 32), jnp.float32),
            pltpu.VMEM_SHARED((N, 32), jnp.float32),
            pltpu.SemaphoreType.DMA((2,)),
        ],
    )
    return scatter(past, fut, cols, rows)


def kernel(x, edge_attr, initial_x, att_edge_attr, params, edge_index):
    rows = edge_index[0]
    cols = edge_index[1]
    tx, ti = _pad_stage(x, initial_x)
    x_i, init_i, x_j, init_j = _gather_stage(tx, ti, cols, rows)

    eu = params["edge_update"]
    fm = params["create_future_msgs"]
    pm = params["create_past_msgs"]
    cb = params["combine_future_past"]

    edge_w = (eu[0][0], eu[0][1], eu[1][0], eu[1][1], eu[2][0], eu[2][1],
              fm[0][0], fm[0][1], fm[1][0], fm[1][1],
              pm[0][0], pm[0][1], pm[1][0], pm[1][1])
    ue, past, fut = _edge_stage(x_i, x_j, init_i, init_j,
                                edge_attr, att_edge_attr, list(edge_w))

    mp, mf = _scatter_stage(past, fut, cols, rows)

    node_w = [cb[0][0][:128], cb[0][0][128:], cb[0][1],
              cb[1][0], cb[1][1], cb[2][0], cb[2][1]]
    updated_nodes = _node_stage(mp, mf, node_w)
    return (updated_nodes, ue)
